# baseline (speedup 1.0000x reference)
import functools

import jax
import jax.numpy as jnp
from jax import lax
from jax.experimental import pallas as pl
from jax.experimental.pallas import tpu as pltpu
from jax.experimental.pallas import tpu_sc as plsc

B = 16384
D = 64
L = 16
NC = 2
NS = 16
NW = NC * NS
CHUNK = B // NW
BK = 128
HALF = CHUNK // 2

_mesh = plsc.VectorSubcoreMesh(core_axis_name="c", subcore_axis_name="s")
_params = pltpu.CompilerParams(needs_layout_passes=False)


def _worker_id():
    return lax.axis_index("s") * NC + lax.axis_index("c")


NBUF = 6


@functools.partial(
    pl.kernel,
    mesh=_mesh,
    out_type=jax.ShapeDtypeStruct((B, BK), jnp.float32),
    compiler_params=_params,
    scratch_types=[
        pltpu.VMEM((CHUNK + L,), jnp.int32),
        pltpu.VMEM((CHUNK,), jnp.int32),
        pltpu.VMEM((NBUF, D, BK), jnp.float32),
        pltpu.VMEM((CHUNK, BK), jnp.float32),
        pltpu.VMEM((CHUNK + 1 + L,), jnp.int32),
        pltpu.VMEM((CHUNK + 1 + L,), jnp.int32),
        [pltpu.SemaphoreType.DMA] * NBUF,
        pltpu.SemaphoreType.DMA,
    ],
)
def _gather_kernel(ut_hbm, ids_hbm, perm_hbm,
                   out_hbm, idxv, posv, tbufs, stag,
                   run_bkt, run_start, ring_sems, sem):
    wid = _worker_id()
    base = wid * CHUNK
    lane16 = lax.iota(jnp.int32, L)
    dvecs = [j * L + lane16 for j in range(4)]

    if True:
        tbl_hbm = ut_hbm
        pltpu.sync_copy(ids_hbm.at[pl.ds(base, CHUNK)],
                        idxv.at[pl.ds(0, CHUNK)])
        pltpu.sync_copy(perm_hbm.at[pl.ds(base, CHUNK)], posv)

        shift1 = jnp.maximum(lane16 - 1, 0)
        lane0 = lane16 == 0

        def scan_body(g, carry):
            prev_last, n = carry
            e0 = g * L
            bkts = lax.shift_right_logical(idxv[pl.ds(e0, L)], 7)
            prevs = jnp.where(lane0, prev_last, jnp.take(bkts, shift1, axis=0))
            mask = bkts != prevs
            plsc.store_compressed(run_bkt.at[pl.ds(n, L)], bkts, mask=mask)
            plsc.store_compressed(run_start.at[pl.ds(n, L)], e0 + lane16, mask=mask)
            n = n + plsc.all_reduce_population_count(mask)[0]
            return bkts[L - 1], n

        _, n = lax.fori_loop(0, CHUNK // L, scan_body,
                             (jnp.int32(-1), jnp.int32(0)))
        plsc.store_compressed(run_start.at[pl.ds(n, L)],
                              jnp.full((L,), CHUNK, jnp.int32), mask=lane0)

        def issue(k, slot):
            bkt = run_bkt[pl.ds(k, L)][0]
            pltpu.async_copy(tbl_hbm.at[:, pl.ds(bkt * BK, BK)],
                             tbufs.at[slot], ring_sems[slot])

        for slot in range(NBUF):
            @pl.when(slot < n)
            def _prime(slot=slot):
                issue(slot, slot)

        def extract(k, slot):
            def ebody(e, _):
                u = idxv[pl.ds(e, L)][0]
                lanev = jnp.full((L,), 0, jnp.int32) + jnp.bitwise_and(u, BK - 1)
                for j in range(4):
                    g = plsc.load_gather(tbufs.at[slot], [dvecs[j], lanev])
                    stag[e, pl.ds(j * L, L)] = g
                return 0
            bounds = run_start[pl.ds(k, L)]
            lax.fori_loop(bounds[0], bounds[1], ebody, 0)

        def group_body(g, _):
            for slot in range(NBUF):
                k = g * NBUF + slot

                @pl.when(k < n)
                def _do(k=k, slot=slot):
                    pltpu.make_async_copy(
                        ut_hbm.at[:, pl.ds(0, BK)], tbufs.at[slot],
                        ring_sems[slot]).wait()
                    extract(k, slot)

                    @pl.when(k + NBUF < n)
                    def _prefetch():
                        issue(k + NBUF, slot)
            return 0

        lax.fori_loop(0, (CHUNK + NBUF - 1) // NBUF, group_body, 0)
        pltpu.async_copy(stag, out_hbm.at[posv], sem).wait()


QC = 128
_BITREV = (0, 8, 4, 12, 2, 10, 6, 14, 1, 9, 5, 13, 3, 11, 7, 15)


@functools.partial(
    pl.kernel,
    mesh=_mesh,
    out_type=jax.ShapeDtypeStruct((B,), jnp.float32),
    compiler_params=_params,
    scratch_types=[
        pltpu.VMEM((2, QC, BK), jnp.float32),
        pltpu.VMEM((2, QC, BK), jnp.float32),
        pltpu.VMEM((CHUNK,), jnp.float32),
        [pltpu.SemaphoreType.DMA] * 2,
        [pltpu.SemaphoreType.DMA] * 2,
    ],
)
def _dot_kernel(ug_hbm, ig_hbm, out_hbm, ub, ib, outv, usems, isems):
    wid = _worker_id()
    base = wid * CHUNK
    lane16 = lax.iota(jnp.int32, L)
    nq = CHUNK // QC

    def issue(q, pp):
        pltpu.async_copy(ug_hbm.at[pl.ds(base + q * QC, QC), :],
                         ub.at[pp], usems[pp])
        pltpu.async_copy(ig_hbm.at[pl.ds(base + q * QC, QC), :],
                         ib.at[pp], isems[pp])

    issue(0, 0)
    for q in range(nq):
        pp = q & 1
        if q + 1 < nq:
            issue(q + 1, 1 - pp)
        pltpu.make_async_copy(ug_hbm.at[pl.ds(0, QC), :], ub.at[pp],
                              usems[pp]).wait()
        pltpu.make_async_copy(ig_hbm.at[pl.ds(0, QC), :], ib.at[pp],
                              isems[pp]).wait()

        def block_body(i, _, pp=pp, q=q):
            vecs = []
            for slot in range(L):
                r = i * L + _BITREV[slot]
                acc = None
                for j in range(4):
                    u = ub[pp, r, pl.ds(j * L, L)]
                    v = ib[pp, r, pl.ds(j * L, L)]
                    t = u * v
                    acc = t if acc is None else acc + t
                vecs.append(acc)
            for st in (8, 4, 2, 1):
                sel = (lane16 & st) == 0
                perm = jnp.bitwise_xor(lane16, st)
                nxt = []
                for a, b in zip(vecs[0::2], vecs[1::2]):
                    a2 = a + jnp.take(a, perm, axis=0)
                    b2 = b + jnp.take(b, perm, axis=0)
                    nxt.append(jnp.where(sel, a2, b2))
                vecs = nxt
            outv[pl.ds(q * QC + i * L, L)] = vecs[0]
            return 0

        lax.fori_loop(0, QC // L, block_body, 0)

    pltpu.sync_copy(outv, out_hbm.at[pl.ds(base, CHUNK)])


def kernel(users, items, user_emb, item_emb):
    users = users.astype(jnp.int32)
    items = items.astype(jnp.int32)
    iota = lax.iota(jnp.int32, B)
    users_s, su = lax.sort_key_val(users, iota)
    items_s, si = lax.sort_key_val(items, iota)
    ug = _gather_kernel(user_emb.T, users_s, su)
    ig = _gather_kernel(item_emb.T, items_s, si)
    return _dot_kernel(ug, ig)


# speedup vs baseline: 1.0116x; 1.0116x over previous
import functools

import jax
import jax.numpy as jnp
from jax import lax
from jax.experimental import pallas as pl
from jax.experimental.pallas import tpu as pltpu
from jax.experimental.pallas import tpu_sc as plsc

B = 16384
D = 64
L = 16
NC = 2
NS = 16
NW = NC * NS
CHUNK = B // NW
BK = 128
HALF = CHUNK // 2

_mesh = plsc.VectorSubcoreMesh(core_axis_name="c", subcore_axis_name="s")
_params = pltpu.CompilerParams(needs_layout_passes=False)


def _worker_id():
    return lax.axis_index("s") * NC + lax.axis_index("c")


NBUF = 7


@functools.partial(
    pl.kernel,
    mesh=_mesh,
    out_type=jax.ShapeDtypeStruct((B, BK), jnp.float32),
    compiler_params=_params,
    scratch_types=[
        pltpu.VMEM((CHUNK + L,), jnp.int32),
        pltpu.VMEM((CHUNK,), jnp.int32),
        pltpu.VMEM((NBUF, D, BK), jnp.float32),
        pltpu.VMEM((CHUNK, BK), jnp.float32),
        pltpu.VMEM((CHUNK + 1 + L,), jnp.int32),
        pltpu.VMEM((CHUNK + 1 + L,), jnp.int32),
        [pltpu.SemaphoreType.DMA] * NBUF,
        pltpu.SemaphoreType.DMA,
    ],
)
def _gather_kernel(ut_hbm, ids_hbm, perm_hbm,
                   out_hbm, idxv, posv, tbufs, stag,
                   run_bkt, run_start, ring_sems, sem):
    wid = _worker_id()
    base = wid * CHUNK
    lane16 = lax.iota(jnp.int32, L)
    dvecs = [j * L + lane16 for j in range(4)]

    if True:
        tbl_hbm = ut_hbm
        pltpu.sync_copy(ids_hbm.at[pl.ds(base, CHUNK)],
                        idxv.at[pl.ds(0, CHUNK)])
        pltpu.sync_copy(perm_hbm.at[pl.ds(base, CHUNK)], posv)

        shift1 = jnp.maximum(lane16 - 1, 0)
        lane0 = lane16 == 0

        def scan_body(g, carry):
            prev_last, n = carry
            e0 = g * L
            bkts = lax.shift_right_logical(idxv[pl.ds(e0, L)], 7)
            prevs = jnp.where(lane0, prev_last, jnp.take(bkts, shift1, axis=0))
            mask = bkts != prevs
            plsc.store_compressed(run_bkt.at[pl.ds(n, L)], bkts, mask=mask)
            plsc.store_compressed(run_start.at[pl.ds(n, L)], e0 + lane16, mask=mask)
            n = n + plsc.all_reduce_population_count(mask)[0]
            return bkts[L - 1], n

        _, n = lax.fori_loop(0, CHUNK // L, scan_body,
                             (jnp.int32(-1), jnp.int32(0)))
        plsc.store_compressed(run_start.at[pl.ds(n, L)],
                              jnp.full((L,), CHUNK, jnp.int32), mask=lane0)

        def issue(k, slot):
            bkt = run_bkt[pl.ds(k, L)][0]
            pltpu.async_copy(tbl_hbm.at[:, pl.ds(bkt * BK, BK)],
                             tbufs.at[slot], ring_sems[slot])

        for slot in range(NBUF):
            @pl.when(slot < n)
            def _prime(slot=slot):
                issue(slot, slot)

        def extract(k, slot):
            def ebody(e, _):
                u = idxv[pl.ds(e, L)][0]
                lanev = jnp.full((L,), 0, jnp.int32) + jnp.bitwise_and(u, BK - 1)
                for j in range(4):
                    g = plsc.load_gather(tbufs.at[slot], [dvecs[j], lanev])
                    stag[e, pl.ds(j * L, L)] = g
                return 0
            bounds = run_start[pl.ds(k, L)]
            lax.fori_loop(bounds[0], bounds[1], ebody, 0)

        def group_body(g, _):
            for slot in range(NBUF):
                k = g * NBUF + slot

                @pl.when(k < n)
                def _do(k=k, slot=slot):
                    pltpu.make_async_copy(
                        ut_hbm.at[:, pl.ds(0, BK)], tbufs.at[slot],
                        ring_sems[slot]).wait()
                    extract(k, slot)

                    @pl.when(k + NBUF < n)
                    def _prefetch():
                        issue(k + NBUF, slot)
            return 0

        lax.fori_loop(0, (CHUNK + NBUF - 1) // NBUF, group_body, 0)
        pltpu.async_copy(stag, out_hbm.at[posv], sem).wait()


QC = 128
_BITREV = (0, 8, 4, 12, 2, 10, 6, 14, 1, 9, 5, 13, 3, 11, 7, 15)


@functools.partial(
    pl.kernel,
    mesh=_mesh,
    out_type=jax.ShapeDtypeStruct((B,), jnp.float32),
    compiler_params=_params,
    scratch_types=[
        pltpu.VMEM((2, QC, BK), jnp.float32),
        pltpu.VMEM((2, QC, BK), jnp.float32),
        pltpu.VMEM((CHUNK,), jnp.float32),
        [pltpu.SemaphoreType.DMA] * 2,
        [pltpu.SemaphoreType.DMA] * 2,
    ],
)
def _dot_kernel(ug_hbm, ig_hbm, out_hbm, ub, ib, outv, usems, isems):
    wid = _worker_id()
    base = wid * CHUNK
    lane16 = lax.iota(jnp.int32, L)
    nq = CHUNK // QC

    def issue(q, pp):
        pltpu.async_copy(ug_hbm.at[pl.ds(base + q * QC, QC), :],
                         ub.at[pp], usems[pp])
        pltpu.async_copy(ig_hbm.at[pl.ds(base + q * QC, QC), :],
                         ib.at[pp], isems[pp])

    issue(0, 0)
    for q in range(nq):
        pp = q & 1
        if q + 1 < nq:
            issue(q + 1, 1 - pp)
        pltpu.make_async_copy(ug_hbm.at[pl.ds(0, QC), :], ub.at[pp],
                              usems[pp]).wait()
        pltpu.make_async_copy(ig_hbm.at[pl.ds(0, QC), :], ib.at[pp],
                              isems[pp]).wait()

        def block_body(i, _, pp=pp, q=q):
            vecs = []
            for slot in range(L):
                r = i * L + _BITREV[slot]
                acc = None
                for j in range(4):
                    u = ub[pp, r, pl.ds(j * L, L)]
                    v = ib[pp, r, pl.ds(j * L, L)]
                    t = u * v
                    acc = t if acc is None else acc + t
                vecs.append(acc)
            for st in (8, 4, 2, 1):
                sel = (lane16 & st) == 0
                perm = jnp.bitwise_xor(lane16, st)
                nxt = []
                for a, b in zip(vecs[0::2], vecs[1::2]):
                    a2 = a + jnp.take(a, perm, axis=0)
                    b2 = b + jnp.take(b, perm, axis=0)
                    nxt.append(jnp.where(sel, a2, b2))
                vecs = nxt
            outv[pl.ds(q * QC + i * L, L)] = vecs[0]
            return 0

        lax.fori_loop(0, QC // L, block_body, 0)

    pltpu.sync_copy(outv, out_hbm.at[pl.ds(base, CHUNK)])


def kernel(users, items, user_emb, item_emb):
    users = users.astype(jnp.int32)
    items = items.astype(jnp.int32)
    iota = lax.iota(jnp.int32, B)
    users_s, su = lax.sort_key_val(users, iota)
    items_s, si = lax.sort_key_val(items, iota)
    ug = _gather_kernel(user_emb.T, users_s, su)
    ig = _gather_kernel(item_emb.T, items_s, si)
    return _dot_kernel(ug, ig)
